# baseline (speedup 1.0000x reference)
import functools

import jax
import jax.numpy as jnp
from jax import lax
from jax.experimental import pallas as pl
from jax.experimental.pallas import tpu as pltpu

ALPHA = 0.9984
R = 256


def _norm_h0_kernel(x_ref, win_ref, xn_ref, h0_ref):
    x = x_ref[...]
    n = jnp.sqrt(jnp.sum(x * x, axis=1, keepdims=True))
    xn_ref[...] = x / jnp.maximum(n, 1e-8)
    h0 = lax.dot_general(x, win_ref[...], (((1,), (0,)), ((), ())),
                         preferred_element_type=jnp.float32)
    h0_ref[...] = jnp.maximum(h0, 0.0)


def _sim_mask_kernel(n_real, np_, nb, k_top, chunk, xnf_ref, xnb_ref, mask_ref,
                     cnt_ref, cacc_ref, eps_ref):
    i = pl.program_id(0)
    rb = xnb_ref.shape[0]
    st = jnp.abs(lax.dot_general(xnf_ref[...], xnb_ref[...],
                                 (((1,), (1,)), ((), ())),
                                 preferred_element_type=jnp.float32))
    cols = i * rb + lax.broadcasted_iota(jnp.int32, (1, rb), 1)
    real = cols < n_real
    kf = float(k_top)

    nc = np_ // chunk
    sr = st.reshape(nc, chunk, rb)
    m1 = jnp.max(sr, axis=1)
    m2 = jnp.max(jnp.where(sr < m1[:, None, :], sr, -1.0), axis=1)

    prev = jnp.full((1, rb), jnp.inf, jnp.float32)
    for _ in range(k_top):
        cand = jnp.where(m1 < prev, m1, jnp.where(m2 < prev, m2, -1.0))
        prev = jnp.max(cand, axis=0, keepdims=True)
    c_last = jnp.sum(jnp.where(st >= prev, 1.0, 0.0), axis=0, keepdims=True)
    eps_ref[...] = jnp.where(real, prev, jnp.inf)
    dup = jnp.where(jnp.logical_and(c_last != kf, real), 1.0, 0.0)

    @pl.when(jnp.max(dup) > 0.0)
    def _():
        prev2 = jnp.full((1, rb), jnp.inf, jnp.float32)
        eps2 = jnp.zeros((1, rb), jnp.float32)
        done = jnp.zeros((1, rb), jnp.float32)
        for _ in range(k_top):
            masked = jnp.where(st < prev2, st, -1.0)
            v = jnp.max(masked, axis=0, keepdims=True)
            c = jnp.sum(jnp.where(st >= v, 1.0, 0.0), axis=0, keepdims=True)
            newly = jnp.where(c >= kf, 1.0, 0.0) * (1.0 - done)
            eps2 = eps2 + newly * v
            done = jnp.minimum(done + newly, 1.0)
            prev2 = v
        eps_ref[...] = jnp.where(dup > 0.0, eps2, eps_ref[...])

    maskf = jnp.where(st >= eps_ref[...], 1.0, 0.0)
    mask_ref[...] = maskf.astype(jnp.int8)

    @pl.when(i == 0)
    def _():
        cacc_ref[...] = jnp.zeros_like(cacc_ref)

    cacc_ref[...] += jnp.sum(maskf, axis=1, keepdims=True)

    @pl.when(i == nb - 1)
    def _():
        cnt_ref[...] = cacc_ref[...]


def _sage_kernel(nb, mask_ref, cnt_ref, hi_ref, hj_ref, wl_ref, wr_ref, b_ref,
                 out_ref, acc_ref):
    i = pl.program_id(1)

    @pl.when(i == 0)
    def _():
        acc_ref[...] = jnp.zeros_like(acc_ref)

    mask = mask_ref[...].astype(jnp.float32)
    acc_ref[...] += lax.dot_general(mask, hi_ref[...],
                                    (((1,), (0,)), ((), ())),
                                    preferred_element_type=jnp.float32)

    @pl.when(i == nb - 1)
    def _():
        agg = acc_ref[...] / jnp.maximum(cnt_ref[...], 1.0)
        o = (lax.dot_general(agg, wl_ref[...], (((1,), (0,)), ((), ())),
                             preferred_element_type=jnp.float32)
             + lax.dot_general(hj_ref[...], wr_ref[...], (((1,), (0,)), ((), ())),
                               preferred_element_type=jnp.float32)
             + b_ref[...])
        out_ref[...] = jnp.maximum(o, 0.0)


def _proj_softmax_kernel(h_ref, w_ref, b_ref, out_ref):
    o = lax.dot_general(h_ref[...], w_ref[...], (((1,), (0,)), ((), ())),
                        preferred_element_type=jnp.float32) + b_ref[...]
    m = jnp.max(o, axis=1, keepdims=True)
    e = jnp.exp(o - m)
    out_ref[...] = e / jnp.sum(e, axis=1, keepdims=True)


RJ = 2048


def _sage_layer(mask, cnt_col, h, wl, wr, b, np_, nb):
    hdim = h.shape[1]
    rj = min(RJ, np_)
    nj = np_ // rj
    return pl.pallas_call(
        functools.partial(_sage_kernel, nb),
        grid=(nj, nb),
        in_specs=[
            pl.BlockSpec((rj, R), lambda j, i: (j, i)),
            pl.BlockSpec((rj, 1), lambda j, i: (j, 0)),
            pl.BlockSpec((R, hdim), lambda j, i: (i, 0)),
            pl.BlockSpec((rj, hdim), lambda j, i: (j, 0)),
            pl.BlockSpec((hdim, hdim), lambda j, i: (0, 0)),
            pl.BlockSpec((hdim, hdim), lambda j, i: (0, 0)),
            pl.BlockSpec((1, hdim), lambda j, i: (0, 0)),
        ],
        out_specs=pl.BlockSpec((rj, hdim), lambda j, i: (j, 0)),
        out_shape=jax.ShapeDtypeStruct((np_, hdim), jnp.float32),
        scratch_shapes=[pltpu.VMEM((rj, hdim), jnp.float32)],
        compiler_params=pltpu.CompilerParams(
            dimension_semantics=("parallel", "arbitrary")),
    )(mask, cnt_col, h, h, wl, wr, b.reshape(1, hdim))


def kernel(x, W_in, Wl1, Wr1, b1, Wl2, Wr2, b2, W_out, b_out):
    n, d_in = x.shape
    hdim = W_in.shape[1]
    d_out = W_out.shape[1]
    np_ = ((n + R - 1) // R) * R
    nb = np_ // R
    k_top = n - int(round(ALPHA * (n - 1)))

    xp = jnp.pad(x, ((0, np_ - n), (0, 0)))

    xn, h0 = pl.pallas_call(
        _norm_h0_kernel,
        grid=(nb,),
        in_specs=[pl.BlockSpec((R, d_in), lambda i: (i, 0)),
                  pl.BlockSpec((d_in, hdim), lambda i: (0, 0))],
        out_specs=[pl.BlockSpec((R, d_in), lambda i: (i, 0)),
                   pl.BlockSpec((R, hdim), lambda i: (i, 0))],
        out_shape=[jax.ShapeDtypeStruct((np_, d_in), jnp.float32),
                   jax.ShapeDtypeStruct((np_, hdim), jnp.float32)],
    )(xp, W_in)

    rb = min(128, np_)
    nb2 = np_ // rb
    mask, cnt_col = pl.pallas_call(
        functools.partial(_sim_mask_kernel, n, np_, nb2, k_top, 8),
        grid=(nb2,),
        in_specs=[pl.BlockSpec((np_, d_in), lambda i: (0, 0)),
                  pl.BlockSpec((rb, d_in), lambda i: (i, 0))],
        out_specs=[pl.BlockSpec((np_, rb), lambda i: (0, i)),
                   pl.BlockSpec((np_, 1), lambda i: (0, 0))],
        out_shape=[jax.ShapeDtypeStruct((np_, np_), jnp.int8),
                   jax.ShapeDtypeStruct((np_, 1), jnp.float32)],
        scratch_shapes=[pltpu.VMEM((np_, 1), jnp.float32),
                        pltpu.VMEM((1, rb), jnp.float32)],
    )(xn, xn)

    h1 = _sage_layer(mask, cnt_col, h0, Wl1, Wr1, b1, np_, nb)
    h2 = _sage_layer(mask, cnt_col, h1, Wl2, Wr2, b2, np_, nb)

    out = pl.pallas_call(
        _proj_softmax_kernel,
        grid=(nb,),
        in_specs=[pl.BlockSpec((R, hdim), lambda i: (i, 0)),
                  pl.BlockSpec((hdim, d_out), lambda i: (0, 0)),
                  pl.BlockSpec((1, d_out), lambda i: (0, 0))],
        out_specs=pl.BlockSpec((R, d_out), lambda i: (i, 0)),
        out_shape=jax.ShapeDtypeStruct((np_, d_out), jnp.float32),
    )(h2, W_out, b_out.reshape(1, d_out))

    return out[:n]


# speedup vs baseline: 2.7754x; 2.7754x over previous
import functools

import jax
import jax.numpy as jnp
from jax import lax
from jax.experimental import pallas as pl
from jax.experimental.pallas import tpu as pltpu

ALPHA = 0.9984
R = 256


def _norm_h0_kernel(x_ref, win_ref, xn_ref, h0_ref):
    x = x_ref[...]
    n = jnp.sqrt(jnp.sum(x * x, axis=1, keepdims=True))
    xn_ref[...] = x / jnp.maximum(n, 1e-8)
    h0 = lax.dot_general(x, win_ref[...], (((1,), (0,)), ((), ())),
                         preferred_element_type=jnp.float32)
    h0_ref[...] = jnp.maximum(h0, 0.0)


def _sim_mask_kernel(n_real, np_, nb, k_top, xnb_ref, xnf_ref, mask_ref,
                     cnt_ref, cacc_ref, eps_ref):
    i = pl.program_id(0)
    s = jnp.abs(lax.dot_general(xnb_ref[...], xnf_ref[...],
                                (((1,), (1,)), ((), ())),
                                preferred_element_type=jnp.float32))
    rows = i * R + lax.broadcasted_iota(jnp.int32, (R, 1), 0)
    real = rows < n_real
    kf = float(k_top)

    sr = s.reshape(R, np_ // 128, 128)
    m1 = jnp.max(sr, axis=1)
    m2 = jnp.max(jnp.where(sr < m1[:, None, :], sr, -1.0), axis=1)
    m3 = jnp.max(jnp.where(sr < m2[:, None, :], sr, -1.0), axis=1)
    m4 = jnp.max(jnp.where(sr < m3[:, None, :], sr, -1.0), axis=1)
    prev = jnp.full((R, 1), jnp.inf, jnp.float32)
    for _ in range(k_top):
        cand = jnp.where(m1 < prev, m1,
                         jnp.where(m2 < prev, m2,
                                   jnp.where(m3 < prev, m3,
                                             jnp.where(m4 < prev, m4, -1.0))))
        prev = jnp.max(cand, axis=1, keepdims=True)
    c_last = jnp.sum(jnp.where(s >= prev, 1.0, 0.0), axis=1, keepdims=True)
    eps_ref[...] = jnp.where(real, prev, jnp.inf)
    dup = jnp.where(jnp.logical_and(c_last != kf, real), 1.0, 0.0)

    @pl.when(jnp.max(dup) > 0.0)
    def _():
        prev2 = jnp.full((R, 1), jnp.inf, jnp.float32)
        eps2 = jnp.zeros((R, 1), jnp.float32)
        done = jnp.zeros((R, 1), jnp.float32)
        for _ in range(k_top):
            masked = jnp.where(s < prev2, s, -1.0)
            v = jnp.max(masked, axis=1, keepdims=True)
            c = jnp.sum(jnp.where(s >= v, 1.0, 0.0), axis=1, keepdims=True)
            newly = jnp.where(c >= kf, 1.0, 0.0) * (1.0 - done)
            eps2 = eps2 + newly * v
            done = jnp.minimum(done + newly, 1.0)
            prev2 = v
        eps_ref[...] = jnp.where(dup > 0.0, eps2, eps_ref[...])

    eps = eps_ref[...]
    maskf = jnp.where(s >= eps, 1.0, 0.0)
    mask_ref[...] = maskf.astype(jnp.int8)

    @pl.when(i == 0)
    def _():
        cacc_ref[...] = jnp.zeros_like(cacc_ref)

    cacc_ref[...] += jnp.sum(maskf, axis=0, keepdims=True)

    @pl.when(i == nb - 1)
    def _():
        cnt_ref[...] = cacc_ref[...]


def _sage_kernel(nb, mask_ref, cnt_ref, hi_ref, hj_ref, wl_ref, wr_ref, b_ref,
                 out_ref, acc_ref):
    i = pl.program_id(1)

    @pl.when(i == 0)
    def _():
        acc_ref[...] = jnp.zeros_like(acc_ref)

    mask = mask_ref[...].astype(jnp.float32)
    acc_ref[...] += lax.dot_general(mask, hi_ref[...],
                                    (((0,), (0,)), ((), ())),
                                    preferred_element_type=jnp.float32)

    @pl.when(i == nb - 1)
    def _():
        agg = acc_ref[...] / jnp.maximum(cnt_ref[...], 1.0)
        o = (lax.dot_general(agg, wl_ref[...], (((1,), (0,)), ((), ())),
                             preferred_element_type=jnp.float32)
             + lax.dot_general(hj_ref[...], wr_ref[...], (((1,), (0,)), ((), ())),
                               preferred_element_type=jnp.float32)
             + b_ref[...])
        out_ref[...] = jnp.maximum(o, 0.0)


def _proj_softmax_kernel(h_ref, w_ref, b_ref, out_ref):
    o = lax.dot_general(h_ref[...], w_ref[...], (((1,), (0,)), ((), ())),
                        preferred_element_type=jnp.float32) + b_ref[...]
    m = jnp.max(o, axis=1, keepdims=True)
    e = jnp.exp(o - m)
    out_ref[...] = e / jnp.sum(e, axis=1, keepdims=True)


RJ = 2048


def _sage_layer(mask, cnt_col, h, wl, wr, b, np_, nb):
    hdim = h.shape[1]
    rj = min(RJ, np_)
    nj = np_ // rj
    return pl.pallas_call(
        functools.partial(_sage_kernel, nb),
        grid=(nj, nb),
        in_specs=[
            pl.BlockSpec((R, rj), lambda j, i: (i, j)),
            pl.BlockSpec((rj, 1), lambda j, i: (j, 0)),
            pl.BlockSpec((R, hdim), lambda j, i: (i, 0)),
            pl.BlockSpec((rj, hdim), lambda j, i: (j, 0)),
            pl.BlockSpec((hdim, hdim), lambda j, i: (0, 0)),
            pl.BlockSpec((hdim, hdim), lambda j, i: (0, 0)),
            pl.BlockSpec((1, hdim), lambda j, i: (0, 0)),
        ],
        out_specs=pl.BlockSpec((rj, hdim), lambda j, i: (j, 0)),
        out_shape=jax.ShapeDtypeStruct((np_, hdim), jnp.float32),
        scratch_shapes=[pltpu.VMEM((rj, hdim), jnp.float32)],
        compiler_params=pltpu.CompilerParams(
            dimension_semantics=("parallel", "arbitrary")),
    )(mask, cnt_col, h, h, wl, wr, b.reshape(1, hdim))


def kernel(x, W_in, Wl1, Wr1, b1, Wl2, Wr2, b2, W_out, b_out):
    n, d_in = x.shape
    hdim = W_in.shape[1]
    d_out = W_out.shape[1]
    np_ = ((n + R - 1) // R) * R
    nb = np_ // R
    k_top = n - int(round(ALPHA * (n - 1)))

    xp = jnp.pad(x, ((0, np_ - n), (0, 0)))

    xn, h0 = pl.pallas_call(
        _norm_h0_kernel,
        grid=(nb,),
        in_specs=[pl.BlockSpec((R, d_in), lambda i: (i, 0)),
                  pl.BlockSpec((d_in, hdim), lambda i: (0, 0))],
        out_specs=[pl.BlockSpec((R, d_in), lambda i: (i, 0)),
                   pl.BlockSpec((R, hdim), lambda i: (i, 0))],
        out_shape=[jax.ShapeDtypeStruct((np_, d_in), jnp.float32),
                   jax.ShapeDtypeStruct((np_, hdim), jnp.float32)],
    )(xp, W_in)

    mask, cnt_row = pl.pallas_call(
        functools.partial(_sim_mask_kernel, n, np_, nb, k_top),
        grid=(nb,),
        in_specs=[pl.BlockSpec((R, d_in), lambda i: (i, 0)),
                  pl.BlockSpec((np_, d_in), lambda i: (0, 0))],
        out_specs=[pl.BlockSpec((R, np_), lambda i: (i, 0)),
                   pl.BlockSpec((1, np_), lambda i: (0, 0))],
        out_shape=[jax.ShapeDtypeStruct((np_, np_), jnp.int8),
                   jax.ShapeDtypeStruct((1, np_), jnp.float32)],
        scratch_shapes=[pltpu.VMEM((1, np_), jnp.float32),
                        pltpu.VMEM((R, 1), jnp.float32)],
    )(xn, xn)
    cnt_col = cnt_row.reshape(np_, 1)

    h1 = _sage_layer(mask, cnt_col, h0, Wl1, Wr1, b1, np_, nb)
    h2 = _sage_layer(mask, cnt_col, h1, Wl2, Wr2, b2, np_, nb)

    out = pl.pallas_call(
        _proj_softmax_kernel,
        grid=(nb,),
        in_specs=[pl.BlockSpec((R, hdim), lambda i: (i, 0)),
                  pl.BlockSpec((hdim, d_out), lambda i: (0, 0)),
                  pl.BlockSpec((1, d_out), lambda i: (0, 0))],
        out_specs=pl.BlockSpec((R, d_out), lambda i: (i, 0)),
        out_shape=jax.ShapeDtypeStruct((np_, d_out), jnp.float32),
    )(h2, W_out, b_out.reshape(1, d_out))

    return out[:n]
